# baseline (speedup 1.0000x reference)
import functools

import jax
import jax.numpy as jnp
from jax import lax
from jax.experimental import pallas as pl
from jax.experimental.pallas import tpu as pltpu
from jax.experimental.pallas import tpu_sc as plsc

B = 16384
EMB = 32
PACK = 128 // EMB
NROW4 = 1000000 // PACK
NC, NS = 2, 16
NW = NC * NS
BPW = B // NW
CHUNK = 128
NCH = BPW // CHUNK


@functools.lru_cache(maxsize=None)
def _make_sc_gather():
  mesh = plsc.VectorSubcoreMesh(
      core_axis_name="c", subcore_axis_name="s", num_cores=NC, num_subcores=NS)

  @functools.partial(
      pl.kernel,
      out_type=[jax.ShapeDtypeStruct((B, 128), jnp.float32)] * 4,
      mesh=mesh,
      scratch_types=[
          pltpu.VMEM((CHUNK,), jnp.int32),
          pltpu.VMEM((CHUNK,), jnp.int32),
          pltpu.VMEM((CHUNK, 128), jnp.float32),
          pltpu.VMEM((CHUNK, 128), jnp.float32),
          pltpu.VMEM((CHUNK, 128), jnp.float32),
          pltpu.VMEM((CHUNK, 128), jnp.float32),
          pltpu.SemaphoreType.DMA,
          pltpu.SemaphoreType.DMA,
      ],
  )
  def sc_gather(u4_h, i4_h, ue_mlp_h, ie_mlp_h, ue_gmf_h, ie_gmf_h,
                out_um_h, out_im_h, out_ug_h, out_ig_h,
                uidx_v, iidx_v, um_v, im_v, ug_v, ig_v, gsem, wsem):
    wid = lax.axis_index("s") * NC + lax.axis_index("c")
    base = wid * BPW

    tables = ((ue_mlp_h, uidx_v, um_v, out_um_h),
              (ie_mlp_h, iidx_v, im_v, out_im_h),
              (ue_gmf_h, uidx_v, ug_v, out_ug_h),
              (ie_gmf_h, iidx_v, ig_v, out_ig_h))

    for c in range(NCH):
      off = base + c * CHUNK
      pltpu.sync_copy(u4_h.at[pl.ds(off, CHUNK)], uidx_v)
      pltpu.sync_copy(i4_h.at[pl.ds(off, CHUNK)], iidx_v)
      gcopies = [pltpu.async_copy(t.at[idx], rows, gsem)
                 for t, idx, rows, _ in tables]
      for gc in gcopies:
        gc.wait()
      wcopies = [pltpu.async_copy(rows, out.at[pl.ds(off, CHUNK)], wsem)
                 for _, _, rows, out in tables]
      for wc in wcopies:
        wc.wait()

  return sc_gather


BLK = 2048


def _extract(rows, onehot):
    acc = onehot[:, 0:1] * rows[:, 0:EMB]
    for q in range(1, PACK):
        acc = acc + onehot[:, q:q + 1] * rows[:, q * EMB:(q + 1) * EMB]
    return acc


def _tc_body(um_ref, im_ref, ug_ref, ig_ref, uoh_ref, ioh_ref,
             w1_ref, b1_ref, w2_ref, b2_ref, wo_ref, bo_ref, out_ref):
    uoh = uoh_ref[...]
    ioh = ioh_ref[...]
    um = _extract(um_ref[...], uoh)
    im = _extract(im_ref[...], ioh)
    ug = _extract(ug_ref[...], uoh)
    ig = _extract(ig_ref[...], ioh)
    h = jnp.dot(um, w1_ref[0:EMB, :], preferred_element_type=jnp.float32)
    h = h + jnp.dot(im, w1_ref[EMB:, :], preferred_element_type=jnp.float32)
    h = jnp.maximum(h + b1_ref[...], 0.0)
    m = jnp.dot(h, w2_ref[...], preferred_element_type=jnp.float32)
    m = jnp.maximum(m + b2_ref[...], 0.0)
    g = ug * ig
    o = jnp.dot(g, wo_ref[0:EMB, :], preferred_element_type=jnp.float32)
    o = o + jnp.dot(m, wo_ref[EMB:, :], preferred_element_type=jnp.float32)
    out_ref[...] = (o + bo_ref[...])[:, 0]


def _tc_dense(um, im, ug, ig, uoh, ioh, W1, b1, W2, b2, Wo, bo):
    grid = (B // BLK,)
    row_spec = pl.BlockSpec((BLK, 128), lambda i: (i, 0))
    oh_spec = pl.BlockSpec((BLK, PACK), lambda i: (i, 0))
    full = lambda shape: pl.BlockSpec(shape, lambda i: (0,) * len(shape))
    return pl.pallas_call(
        _tc_body,
        grid=grid,
        in_specs=[row_spec, row_spec, row_spec, row_spec, oh_spec, oh_spec,
                  full((2 * EMB, 64)), full((1, 64)),
                  full((64, EMB)), full((1, EMB)),
                  full((2 * EMB, 1)), full((1, 1))],
        out_specs=pl.BlockSpec((BLK,), lambda i: (i,)),
        out_shape=jax.ShapeDtypeStruct((B,), jnp.float32),
        compiler_params=pltpu.CompilerParams(
            dimension_semantics=("arbitrary",)),
    )(um, im, ug, ig, uoh, ioh, W1, b1, W2, b2, Wo, bo)


def kernel(user, item, user_emb_mlp, item_emb_mlp, user_emb_gmf, item_emb_gmf,
           W1, b1, W2, b2, Wo, bo):
    user = user.astype(jnp.int32)
    item = item.astype(jnp.int32)
    u4 = user // PACK
    i4 = item // PACK
    uoh = jax.nn.one_hot(user % PACK, PACK, dtype=jnp.float32)
    ioh = jax.nn.one_hot(item % PACK, PACK, dtype=jnp.float32)
    t4 = [t.reshape(NROW4, 128) for t in
          (user_emb_mlp, item_emb_mlp, user_emb_gmf, item_emb_gmf)]
    um, im, ug, ig = _make_sc_gather()(u4, i4, *t4)
    return _tc_dense(um, im, ug, ig, uoh, ioh,
                     W1, b1.reshape(1, -1), W2, b2.reshape(1, -1),
                     Wo, bo.reshape(1, 1))


# speedup vs baseline: 2.2177x; 2.2177x over previous
import functools

import jax
import jax.numpy as jnp
from jax import lax
from jax.experimental import pallas as pl
from jax.experimental.pallas import tpu as pltpu
from jax.experimental.pallas import tpu_sc as plsc

B = 16384
EMB = 32
NTILE = 1000000 // 8
NC, NS = 2, 16
NW = NC * NS
BPW = B // NW
CH = 16
NCH = BPW // CH
PACK = 4
BP = B // PACK


@functools.lru_cache(maxsize=None)
def _make_sc_gather():
  mesh = plsc.VectorSubcoreMesh(
      core_axis_name="c", subcore_axis_name="s", num_cores=NC, num_subcores=NS)

  @functools.partial(
      pl.kernel,
      out_type=[jax.ShapeDtypeStruct((B * EMB,), jnp.float32)] * 4,
      mesh=mesh,
      scratch_types=[
          pltpu.SMEM((BPW,), jnp.int32),
          pltpu.SMEM((BPW,), jnp.int32),
          pltpu.VMEM((BPW,), jnp.int32),
          pltpu.VMEM((CH, 8, EMB), jnp.float32),
          pltpu.VMEM((CH, 8, EMB), jnp.float32),
          pltpu.VMEM((CH, 8, EMB), jnp.float32),
          pltpu.VMEM((CH, 8, EMB), jnp.float32),
          pltpu.VMEM((CH * EMB,), jnp.float32),
          pltpu.VMEM((CH * EMB,), jnp.float32),
          pltpu.VMEM((CH * EMB,), jnp.float32),
          pltpu.VMEM((CH * EMB,), jnp.float32),
          pltpu.SemaphoreType.DMA,
          pltpu.SemaphoreType.DMA,
      ],
      compiler_params=pltpu.CompilerParams(needs_layout_passes=False),
  )
  def sc_gather(user_h, item_h,
                ue_mlp_h, ie_mlp_h, ue_gmf_h, ie_gmf_h,
                out_um_h, out_im_h, out_ug_h, out_ig_h,
                uidx_s, iidx_s, idx_v,
                um_t, im_t, ug_t, ig_t,
                um_v, im_v, ug_v, ig_v, gsem, wsem):
    wid = lax.axis_index("s") * NC + lax.axis_index("c")
    base = wid * BPW
    lane = lax.iota(jnp.int32, 16)

    def stage_scalars(dst_s):
      def body16(k, carry):
        v = idx_v[pl.ds(k * 16, 16)].astype(jnp.float32)
        for q in range(16):
          dst_s[k * 16 + q] = jnp.sum(
              jnp.where(lane == q, v, 0.0)).astype(jnp.int32)
        return carry
      lax.fori_loop(0, BPW // 16, body16, None)

    pltpu.sync_copy(user_h.at[pl.ds(base, BPW)], idx_v)
    stage_scalars(uidx_s)
    pltpu.sync_copy(item_h.at[pl.ds(base, BPW)], idx_v)
    stage_scalars(iidx_s)

    tabs = ((ue_mlp_h, None, um_t, um_v, uidx_s, out_um_h),
            (ie_mlp_h, None, im_t, im_v, iidx_s, out_im_h),
            (ue_gmf_h, None, ug_t, ug_v, uidx_s, out_ug_h),
            (ie_gmf_h, None, ig_t, ig_v, iidx_s, out_ig_h))

    def chunk(c, _):
      off = base + c * CH

      def fire(j, carry):
        for t, _tid, tiles, _rows, idx_s, _o in tabs:
          tid = idx_s[c * CH + j] >> 3
          pltpu.async_copy(t.at[tid], tiles.at[j], gsem)
        return carry
      lax.fori_loop(0, CH, fire, None)
      for t, _tid, tiles, _rows, _s, _o in tabs:
        pltpu.make_async_copy(t.at[pl.ds(0, CH)], tiles, gsem).wait()

      def row(j, carry):
        for _t, _tid, tiles, rows, idx_s, _o in tabs:
          r = idx_s[c * CH + j] & 7
          for half in range(2):
            rows[pl.ds(j * EMB + half * 16, 16)] = (
                tiles[j, r, pl.ds(half * 16, 16)])
        return carry
      lax.fori_loop(0, CH, row, None)
      wcopies = [pltpu.async_copy(
          rows, out.at[pl.ds(off * EMB, CH * EMB)], wsem)
          for _, _, _, rows, _, out in tabs]
      for wc in wcopies:
        wc.wait()
      return _

    lax.fori_loop(0, NCH, chunk, None)

  return sc_gather


BLK = 1024


def _tc_body(um_ref, im_ref, ug_ref, ig_ref,
             w1u_ref, w1i_ref, b1_ref, w2_ref, b2_ref,
             wog_ref, wom_ref, bo_ref, out_ref):
    h = jnp.dot(um_ref[...], w1u_ref[...], preferred_element_type=jnp.float32)
    h = h + jnp.dot(im_ref[...], w1i_ref[...], preferred_element_type=jnp.float32)
    h = jnp.maximum(h + b1_ref[...], 0.0)
    m = jnp.dot(h, w2_ref[...], preferred_element_type=jnp.float32)
    m = jnp.maximum(m + b2_ref[...], 0.0)
    g = ug_ref[...] * ig_ref[...]
    o = jnp.dot(g, wog_ref[...], preferred_element_type=jnp.float32)
    o = o + jnp.dot(m, wom_ref[...], preferred_element_type=jnp.float32)
    out_ref[...] = o + bo_ref[...]


def _tc_dense(um, im, ug, ig, W1u_e, W1i_e, b1_e, W2_e, b2_e, Wog_e, Wom_e, bo):
    grid = (BP // BLK,)
    row_spec = pl.BlockSpec((BLK, PACK * EMB), lambda i: (i, 0))
    full = lambda shape: pl.BlockSpec(shape, lambda i: (0,) * len(shape))
    return pl.pallas_call(
        _tc_body,
        grid=grid,
        in_specs=[row_spec, row_spec, row_spec, row_spec,
                  full((PACK * EMB, PACK * 64)), full((PACK * EMB, PACK * 64)),
                  full((1, PACK * 64)),
                  full((PACK * 64, PACK * EMB)), full((1, PACK * EMB)),
                  full((PACK * EMB, PACK)), full((PACK * EMB, PACK)),
                  full((1, 1))],
        out_specs=pl.BlockSpec((BLK, PACK), lambda i: (i, 0)),
        out_shape=jax.ShapeDtypeStruct((BP, PACK), jnp.float32),
        compiler_params=pltpu.CompilerParams(
            dimension_semantics=("arbitrary",)),
    )(um, im, ug, ig, W1u_e, W1i_e, b1_e, W2_e, b2_e, Wog_e, Wom_e, bo)


def _block_diag(w):
    r, c = w.shape
    eye = jnp.eye(PACK, dtype=w.dtype)
    return (eye[:, None, :, None] * w[None, :, None, :]).reshape(PACK * r, PACK * c)


def kernel(user, item, user_emb_mlp, item_emb_mlp, user_emb_gmf, item_emb_gmf,
           W1, b1, W2, b2, Wo, bo):
    user = user.astype(jnp.int32)
    item = item.astype(jnp.int32)
    t3 = [t.reshape(NTILE, 8, EMB) for t in
          (user_emb_mlp, item_emb_mlp, user_emb_gmf, item_emb_gmf)]
    um, im, ug, ig = _make_sc_gather()(user, item, *t3)
    um = um.reshape(BP, PACK * EMB)
    im = im.reshape(BP, PACK * EMB)
    ug = ug.reshape(BP, PACK * EMB)
    ig = ig.reshape(BP, PACK * EMB)
    W1u_e = _block_diag(W1[:EMB])
    W1i_e = _block_diag(W1[EMB:])
    W2_e = _block_diag(W2)
    Wog_e = _block_diag(Wo[:EMB])
    Wom_e = _block_diag(Wo[EMB:])
    b1_e = jnp.tile(b1, PACK).reshape(1, -1)
    b2_e = jnp.tile(b2, PACK).reshape(1, -1)
    out_p = _tc_dense(um, im, ug, ig, W1u_e, W1i_e, b1_e, W2_e, b2_e,
                      Wog_e, Wom_e, bo.reshape(1, 1))
    return out_p.reshape(B)
